# baseline (speedup 1.0000x reference)
import jax
import jax.numpy as jnp
from jax.experimental import pallas as pl
from jax.experimental.pallas import tpu as pltpu

_TILES = (1024, 2048, 2048, 2048, 2048, 2048, 2048, 2048, 512, 512)
_MAXT = max(_TILES)
_NBUF = 4


def _offsets(tiles):
    offs, o = [], 0
    for t in tiles:
        offs.append(o)
        o += t
    return tuple(offs)


_OFFS = _offsets(_TILES)


def _matmul_kernel(x_hbm, mem_hbm, out_hbm, xv_ref, mbufs, obufs,
                   xsem, rsems, wsems):
    nt = len(_TILES)

    def read(i):
        sz, off = _TILES[i], _OFFS[i]
        return pltpu.make_async_copy(
            mem_hbm.at[pl.ds(off, sz), :],
            mbufs.at[i % _NBUF, pl.ds(0, sz), :],
            rsems.at[i % _NBUF])

    def write(i):
        sz, off = _TILES[i], _OFFS[i]
        return pltpu.make_async_copy(
            obufs.at[i % _NBUF, :, pl.ds(0, sz)],
            out_hbm.at[:, pl.ds(off, sz)],
            wsems.at[i % _NBUF])

    x_copy = pltpu.make_async_copy(x_hbm, xv_ref, xsem)
    x_copy.start()
    for i in range(min(_NBUF, nt)):
        read(i).start()
    x_copy.wait()

    for i in range(nt):
        sz = _TILES[i]
        read(i).wait()
        if i >= _NBUF:
            write(i - _NBUF).wait()
        mb = mbufs[i % _NBUF, pl.ds(0, sz), :]
        obufs[i % _NBUF, :, pl.ds(0, sz)] = jax.lax.dot_general(
            xv_ref[...], mb, (((1,), (1,)), ((), ())),
            precision=jax.lax.Precision.DEFAULT,
            preferred_element_type=jnp.float32)
        write(i).start()
        if i + _NBUF < nt:
            read(i + _NBUF).start()

    for i in range(max(nt - _NBUF, 0), nt):
        write(i).wait()


def kernel(x, targets, memory):
    del targets
    b, d = x.shape
    n = memory.shape[0]
    return pl.pallas_call(
        _matmul_kernel,
        in_specs=[
            pl.BlockSpec(memory_space=pltpu.MemorySpace.HBM),
            pl.BlockSpec(memory_space=pltpu.MemorySpace.HBM),
        ],
        out_specs=pl.BlockSpec(memory_space=pltpu.MemorySpace.HBM),
        out_shape=jax.ShapeDtypeStruct((b, n), jnp.float32),
        scratch_shapes=[
            pltpu.VMEM((b, d), jnp.float32),
            pltpu.VMEM((_NBUF, _MAXT, d), jnp.float32),
            pltpu.VMEM((_NBUF, b, _MAXT), jnp.float32),
            pltpu.SemaphoreType.DMA,
            pltpu.SemaphoreType.DMA((_NBUF,)),
            pltpu.SemaphoreType.DMA((_NBUF,)),
        ],
    )(x, memory)


# speedup vs baseline: 1.0005x; 1.0005x over previous
import jax
import jax.numpy as jnp
from jax.experimental import pallas as pl
from jax.experimental.pallas import tpu as pltpu

_TILES = (1024, 2048, 2048, 2048, 2048, 2048, 2048, 2048, 1024)
_MAXT = max(_TILES)
_NRBUF = 5
_NWBUF = 4


def _offsets(tiles):
    offs, o = [], 0
    for t in tiles:
        offs.append(o)
        o += t
    return tuple(offs)


_OFFS = _offsets(_TILES)


def _matmul_kernel(x_hbm, mem_hbm, out_hbm, xv_ref, mbufs, obufs,
                   xsem, rsems, wsems):
    nt = len(_TILES)

    def read(i):
        sz, off = _TILES[i], _OFFS[i]
        return pltpu.make_async_copy(
            mem_hbm.at[pl.ds(off, sz), :],
            mbufs.at[i % _NRBUF, pl.ds(0, sz), :],
            rsems.at[i % _NRBUF])

    def write(i):
        sz, off = _TILES[i], _OFFS[i]
        return pltpu.make_async_copy(
            obufs.at[i % _NWBUF, :, pl.ds(0, sz)],
            out_hbm.at[:, pl.ds(off, sz)],
            wsems.at[i % _NWBUF])

    x_copy = pltpu.make_async_copy(x_hbm, xv_ref, xsem)
    x_copy.start()
    for i in range(min(_NRBUF, nt)):
        read(i).start()
    x_copy.wait()

    for i in range(nt):
        sz = _TILES[i]
        read(i).wait()
        if i >= _NWBUF:
            write(i - _NWBUF).wait()
        mb = mbufs[i % _NRBUF, pl.ds(0, sz), :]
        obufs[i % _NWBUF, :, pl.ds(0, sz)] = jax.lax.dot_general(
            xv_ref[...], mb, (((1,), (1,)), ((), ())),
            precision=jax.lax.Precision.DEFAULT,
            preferred_element_type=jnp.float32)
        write(i).start()
        if i + _NRBUF < nt:
            read(i + _NRBUF).start()

    for i in range(max(nt - _NWBUF, 0), nt):
        write(i).wait()


def kernel(x, targets, memory):
    del targets
    b, d = x.shape
    n = memory.shape[0]
    return pl.pallas_call(
        _matmul_kernel,
        in_specs=[
            pl.BlockSpec(memory_space=pltpu.MemorySpace.HBM),
            pl.BlockSpec(memory_space=pltpu.MemorySpace.HBM),
        ],
        out_specs=pl.BlockSpec(memory_space=pltpu.MemorySpace.HBM),
        out_shape=jax.ShapeDtypeStruct((b, n), jnp.float32),
        scratch_shapes=[
            pltpu.VMEM((b, d), jnp.float32),
            pltpu.VMEM((_NRBUF, _MAXT, d), jnp.float32),
            pltpu.VMEM((_NWBUF, b, _MAXT), jnp.float32),
            pltpu.SemaphoreType.DMA,
            pltpu.SemaphoreType.DMA((_NRBUF,)),
            pltpu.SemaphoreType.DMA((_NWBUF,)),
        ],
    )(x, memory)
